# baseline (device time: 48926 ns/iter reference)
import jax
import jax.numpy as jnp
from jax import lax
from jax.experimental import pallas as pl
from jax.experimental.pallas import tpu as pltpu

N_DEV = 8


def kernel(Q, K, V):
    b, sq, h, d = Q.shape
    skv = K.shape[1]
    scale = d ** -0.5
    bh = b * h
    rows = b * skv * h

    def body(q_ref, k_ref, v_ref, out_ref, mine_ref, comm_ref, send_sems, recv_sems):
        my = lax.axis_index("i")

        k2 = k_ref[:].reshape(rows, d).astype(jnp.bfloat16)
        q2 = (q_ref[:].reshape(bh, d) * scale).astype(jnp.bfloat16)
        s3 = lax.dot_general(
            k2, q2, (((1,), (1,)), ((), ())), preferred_element_type=jnp.float32
        )

        hm = (
            lax.broadcasted_iota(jnp.int32, (h, bh), 0)
            == lax.broadcasted_iota(jnp.int32, (h, bh), 1) % h
        ).astype(jnp.float32)
        ssum = (s3.reshape(b * skv, h, bh) * hm[None]).sum(axis=1)

        bm = (
            lax.broadcasted_iota(jnp.int32, (b * skv, bh), 0) // skv
            == lax.broadcasted_iota(jnp.int32, (b * skv, bh), 1) // h
        ).astype(jnp.float32)
        p = jnp.exp(ssum) * bm
        den = p.sum(axis=0)

        p3 = (p[:, None, :] * hm[None]).reshape(rows, bh).astype(jnp.bfloat16)
        v2 = v_ref[:].reshape(rows, d).astype(jnp.bfloat16)
        o2 = lax.dot_general(
            p3, v2, (((0,), (0,)), ((), ())), preferred_element_type=jnp.float32
        )

        mine_ref[0] = o2
        mine_ref[1] = jnp.broadcast_to(den[:, None], (bh, d))

        rdmas = []
        for off in range(1, N_DEV):
            tgt = (my + off) % N_DEV
            rdma = pltpu.make_async_remote_copy(
                src_ref=mine_ref,
                dst_ref=comm_ref.at[off - 1],
                send_sem=send_sems.at[off - 1],
                recv_sem=recv_sems.at[off - 1],
                device_id=(tgt,),
                device_id_type=pl.DeviceIdType.MESH,
            )
            rdma.start()
            rdmas.append(rdma)

        for rdma in rdmas:
            rdma.wait_recv()

        acc = mine_ref[:]
        for slot in range(N_DEV - 1):
            acc = acc + comm_ref[slot]
        out_ref[:] = (acc[0] / acc[1]).reshape(b, sq, h, d)

        for rdma in rdmas:
            rdma.wait_send()

    return pl.pallas_call(
        body,
        out_shape=jax.ShapeDtypeStruct((b, sq, h, d), jnp.float32),
        in_specs=[
            pl.BlockSpec(memory_space=pltpu.VMEM),
            pl.BlockSpec(memory_space=pltpu.VMEM),
            pl.BlockSpec(memory_space=pltpu.VMEM),
        ],
        out_specs=pl.BlockSpec(memory_space=pltpu.VMEM),
        scratch_shapes=[
            pltpu.VMEM((2, bh, d), jnp.float32),
            pltpu.VMEM((N_DEV - 1, 2, bh, d), jnp.float32),
            pltpu.SemaphoreType.DMA((N_DEV - 1,)),
            pltpu.SemaphoreType.DMA((N_DEV - 1,)),
        ],
    )(Q, K, V)


# device time: 21292 ns/iter; 2.2979x vs baseline; 2.2979x over previous
import jax
import jax.numpy as jnp
from jax import lax
from jax.experimental import pallas as pl
from jax.experimental.pallas import tpu as pltpu

N_DEV = 8


def kernel(Q, K, V):
    b, sq, h, d = Q.shape
    skv = K.shape[1]
    scale = d ** -0.5
    bh = b * h
    bhd = bh * d

    Qf = Q.reshape(bh, d)
    Kt = jnp.transpose(K, (0, 2, 3, 1)).reshape(bhd, skv)
    Vt = jnp.transpose(V, (0, 2, 3, 1)).reshape(bhd, skv)

    def body(q_ref, k_ref, v_ref, out_ref, mine_ref, comm_ref, send_sems, recv_sems):
        my = lax.axis_index("i")

        q2 = q_ref[:] * scale
        qt = jnp.broadcast_to(q2[:, None, :], (bh, bh, d)).reshape(bh, bhd)
        mbd = (
            lax.broadcasted_iota(jnp.int32, (bh, bhd), 1) // d
            == lax.broadcasted_iota(jnp.int32, (bh, bhd), 0)
        )
        qbd = jnp.where(mbd, qt, 0.0).astype(jnp.bfloat16)

        s = lax.dot_general(
            qbd, k_ref[:].astype(jnp.bfloat16),
            (((1,), (0,)), ((), ())),
            preferred_element_type=jnp.float32,
        )
        p = jnp.exp(s)
        den = jnp.sum(p, axis=1, keepdims=True)

        p3 = jnp.broadcast_to(p[:, None, :], (bh, d, skv)).reshape(bhd, skv)
        o2 = jnp.sum(v_ref[:] * p3, axis=1).reshape(bh, d)

        packed = jnp.concatenate(
            [o2, jnp.broadcast_to(den, (bh, d))], axis=1
        ).astype(jnp.bfloat16)
        mine_ref[:] = packed

        rdmas = []
        for off in range(1, N_DEV):
            tgt = (my + off) % N_DEV
            rdma = pltpu.make_async_remote_copy(
                src_ref=mine_ref,
                dst_ref=comm_ref.at[off - 1],
                send_sem=send_sems.at[off - 1],
                recv_sem=recv_sems.at[off - 1],
                device_id=(tgt,),
                device_id_type=pl.DeviceIdType.MESH,
            )
            rdma.start()
            rdmas.append(rdma)

        for rdma in rdmas:
            rdma.wait_recv()

        acc = mine_ref[:].astype(jnp.float32)
        for slot in range(N_DEV - 1):
            acc = acc + comm_ref[slot].astype(jnp.float32)
        out = acc[:, :d] / acc[:, d:]
        out_ref[:] = out.reshape(b, sq, h, d)

        for rdma in rdmas:
            rdma.wait_send()

    return pl.pallas_call(
        body,
        out_shape=jax.ShapeDtypeStruct((b, sq, h, d), jnp.float32),
        in_specs=[pl.BlockSpec(memory_space=pltpu.VMEM)] * 3,
        out_specs=pl.BlockSpec(memory_space=pltpu.VMEM),
        scratch_shapes=[
            pltpu.VMEM((bh, 2 * d), jnp.bfloat16),
            pltpu.VMEM((N_DEV - 1, bh, 2 * d), jnp.bfloat16),
            pltpu.SemaphoreType.DMA((N_DEV - 1,)),
            pltpu.SemaphoreType.DMA((N_DEV - 1,)),
        ],
    )(Qf, Kt, Vt)
